# baseline (speedup 1.0000x reference)
import jax
import jax.numpy as jnp
from jax.experimental import pallas as pl
from jax.experimental.pallas import tpu as pltpu

G = 24
NPG = 256
KNN = 80
NF = 7
NI = 128
NI2 = 6 * NI
NL = 3
N = G * NPG

_NEG_SLOPE = 0.01


def _leaky(x):
    return jnp.where(x >= 0, x, x * jnp.float32(_NEG_SLOPE))


def _body(x_ref, pos_ref, post_ref, w1_ref, b1_ref, w2_ref, b2_ref, w3_ref,
          b3_ref, bng_ref, bnb_ref, lb_ref, ow_ref, ob_ref, lw_hbm,
          out_ref, lw_vmem, adj_ref, z_ref, dma_sem):
    f32 = jnp.float32
    head_cp = pltpu.make_async_copy(lw_hbm, lw_vmem, dma_sem)
    head_cp.start()

    d2 = jnp.zeros((G, NPG, NPG), f32)
    for c in range(4):
        col = pos_ref[:, :, c:c + 1]
        row = post_ref[:, c:c + 1, :]
        diff = col - row
        d2 = d2 + diff * diff
    ii = jax.lax.broadcasted_iota(jnp.int32, (G, NPG, NPG), 1)
    jj = jax.lax.broadcasted_iota(jnp.int32, (G, NPG, NPG), 2)
    d2 = d2 + jnp.where(ii == jj, f32(1e12), f32(0.0))

    bits = jax.lax.bitcast_convert_type(d2, jnp.int32)
    one = f32(1.0)
    zero = f32(0.0)
    half = NPG // 2

    def search_step(it, thr):
        cand = thr | jnp.left_shift(jnp.int32(1), 30 - it)
        ltf = (jnp.where(bits[:, :half, :] < cand, one, zero)
               + jnp.where(bits[:, half:, :] < cand, one, zero))
        cnt = jnp.sum(ltf, axis=1, keepdims=True)
        return jnp.where(cnt <= f32(KNN - 1), cand, thr)

    thr = jax.lax.fori_loop(0, 31, search_step,
                            jnp.zeros((G, 1, NPG), jnp.int32))

    ltm = bits < thr
    eqm = bits == thr
    cnt_lt = (jnp.sum(jnp.where(ltm[:, :half, :], one, zero)
                      + jnp.where(ltm[:, half:, :], one, zero),
                      axis=1, keepdims=True))
    need = f32(KNN) - cnt_lt
    jj2 = jax.lax.broadcasted_iota(jnp.int32, (NPG, NPG), 0)
    kk2 = jax.lax.broadcasted_iota(jnp.int32, (NPG, NPG), 1)
    lower_tri = jnp.where(kk2 < jj2, one, zero).astype(jnp.bfloat16)
    eqb = jnp.where(eqm, one, zero).astype(jnp.bfloat16)
    tie_sel = []
    for g in range(G):
        eq_rank = jnp.dot(lower_tri, eqb[g], preferred_element_type=f32)
        tie_sel.append(eq_rank < need[g])
    tie = jnp.stack(tie_sel, axis=0)
    sel = ltm | (eqm & tie)
    adj_ref[...] = jnp.where(sel, one, zero).astype(
        jnp.bfloat16).reshape(N, NPG)

    dn = (((0,), (0,)), ((), ()))

    def agg(at, v):
        hi = v.astype(jnp.bfloat16)
        lo = (v - hi.astype(f32)).astype(jnp.bfloat16)
        s = (jax.lax.dot_general(at, hi, dn, preferred_element_type=f32)
             + jax.lax.dot_general(at, lo, dn, preferred_element_type=f32))
        return s * f32(1.0 / KNN)

    def graph_step(gg, carry):
        for u in range(3):
            g = gg * 3 + u
            at = adj_ref[pl.ds(g * NPG, NPG), :]
            h = x_ref[pl.ds(g * NPG, NPG), :]
            feats = []
            for (w_ref, b_ref) in ((w1_ref, b1_ref), (w2_ref, b2_ref),
                                   (w3_ref, b3_ref)):
                a1 = agg(at, h)
                a2 = agg(at, a1)
                o = jnp.dot(h, w_ref[0], preferred_element_type=f32)
                o = o + jnp.dot(a1, w_ref[1], preferred_element_type=f32)
                o = o + jnp.dot(a2, w_ref[2], preferred_element_type=f32)
                h = _leaky(o + b_ref[...])
                mean = jnp.sum(h, axis=0, keepdims=True) * f32(1.0 / NPG)
                mx = jnp.max(h, axis=0, keepdims=True)
                feats.append(mean)
                feats.append(mx)
            z_ref[g, :, :] = jnp.concatenate(feats, axis=1)
        return carry

    jax.lax.fori_loop(0, G // 3, graph_step, jnp.int32(0))

    inv = one / jnp.sqrt(f32(1.0 + 1e-5))
    z = z_ref[...].reshape(G, NI2) * inv * bng_ref[...] + bnb_ref[...]
    head_cp.wait()
    for i in range(5):
        z = _leaky(jnp.dot(z, lw_vmem[i], preferred_element_type=f32)
                   + lb_ref[i])
    out_ref[...] = jnp.dot(z, ow_ref[...], preferred_element_type=f32) \
        + ob_ref[...]


def kernel(x, batch, params):
    f32 = jnp.float32
    x8 = jnp.pad(x, ((0, 0), (0, 1)))
    pos3 = x[:, :4].reshape(G, NPG, 4)
    post = pos3.transpose(0, 2, 1)
    w1 = jnp.pad(params['conv1_w'], ((0, 0), (0, 1), (0, 0)))

    vmem = lambda: pl.BlockSpec(memory_space=pltpu.VMEM)
    out = pl.pallas_call(
        _body,
        in_specs=[vmem() for _ in range(14)] +
                 [pl.BlockSpec(memory_space=pl.ANY)],
        out_specs=vmem(),
        out_shape=jax.ShapeDtypeStruct((G, NL), f32),
        scratch_shapes=[
            pltpu.VMEM((5, NI2, NI2), f32),
            pltpu.VMEM((N, NPG), jnp.bfloat16),
            pltpu.VMEM((G, 1, NI2), f32),
            pltpu.SemaphoreType.DMA,
        ],
    )(
        x8, pos3, post, w1, params['conv1_b'].reshape(1, NI),
        params['conv2_w'], params['conv2_b'].reshape(1, NI),
        params['conv3_w'], params['conv3_b'].reshape(1, NI),
        params['bn_g'].reshape(1, NI2), params['bn_b'].reshape(1, NI2),
        params['lin_b'].reshape(5, 1, NI2), params['out_w'],
        params['out_b'].reshape(1, NL), params['lin_w'],
    )
    return out


# speedup vs baseline: 1.0472x; 1.0472x over previous
import jax
import jax.numpy as jnp
from jax.experimental import pallas as pl
from jax.experimental.pallas import tpu as pltpu

G = 24
NPG = 256
KNN = 80
NF = 7
NI = 128
NI2 = 6 * NI
NL = 3
N = G * NPG

_NEG_SLOPE = 0.01


def _leaky(x):
    return jnp.where(x >= 0, x, x * jnp.float32(_NEG_SLOPE))


def _body(x_ref, pos_ref, post_ref, w1_ref, b1_ref, w2_ref, b2_ref, w3_ref,
          b3_ref, bng_ref, bnb_ref, lb_ref, ow_ref, ob_ref, lw_hbm,
          out_ref, lw_vmem, adj_ref, z_ref, dma_sem):
    f32 = jnp.float32
    head_cp = pltpu.make_async_copy(lw_hbm, lw_vmem, dma_sem)
    head_cp.start()

    d2 = jnp.zeros((G, NPG, NPG), f32)
    for c in range(4):
        col = pos_ref[:, :, c:c + 1]
        row = post_ref[:, c:c + 1, :]
        diff = col - row
        d2 = d2 + diff * diff
    ii = jax.lax.broadcasted_iota(jnp.int32, (G, NPG, NPG), 1)
    jj = jax.lax.broadcasted_iota(jnp.int32, (G, NPG, NPG), 2)
    d2 = d2 + jnp.where(ii == jj, f32(1e12), f32(0.0))

    bits = jax.lax.bitcast_convert_type(d2, jnp.int32)
    one = f32(1.0)
    zero = f32(0.0)
    half = NPG // 2

    def search_step(it, thr):
        cand = thr | jnp.left_shift(jnp.int32(1), 30 - it)
        ltf = (jnp.where(bits[:, :half, :] < cand, one, zero)
               + jnp.where(bits[:, half:, :] < cand, one, zero))
        cnt = jnp.sum(ltf, axis=1, keepdims=True)
        return jnp.where(cnt <= f32(KNN - 1), cand, thr)

    thr = jax.lax.fori_loop(0, 31, search_step,
                            jnp.zeros((G, 1, NPG), jnp.int32))

    ltm = bits < thr
    eqm = bits == thr
    cnt_lt = (jnp.sum(jnp.where(ltm[:, :half, :], one, zero)
                      + jnp.where(ltm[:, half:, :], one, zero),
                      axis=1, keepdims=True))
    need = f32(KNN) - cnt_lt
    jj2 = jax.lax.broadcasted_iota(jnp.int32, (NPG, NPG), 0)
    kk2 = jax.lax.broadcasted_iota(jnp.int32, (NPG, NPG), 1)
    lower_tri = jnp.where(kk2 < jj2, one, zero).astype(jnp.bfloat16)
    eqb = jnp.where(eqm, one, zero).astype(jnp.bfloat16)
    tie_sel = []
    for g in range(G):
        eq_rank = jnp.dot(lower_tri, eqb[g], preferred_element_type=f32)
        tie_sel.append(eq_rank < need[g])
    tie = jnp.stack(tie_sel, axis=0)
    sel = ltm | (eqm & tie)
    adj_ref[...] = jnp.where(sel, f32(1.0 / KNN), zero).reshape(N, NPG)

    dn = (((0,), (0,)), ((), ()))

    def graph_step(gg, carry):
        for u in range(2):
            g = gg * 2 + u
            at = adj_ref[pl.ds(g * NPG, NPG), :]
            h = x_ref[pl.ds(g * NPG, NPG), :]
            feats = []
            for (w_ref, b_ref) in ((w1_ref, b1_ref), (w2_ref, b2_ref),
                                   (w3_ref, b3_ref)):
                a1 = jax.lax.dot_general(at, h, dn,
                                         preferred_element_type=f32)
                a2 = jax.lax.dot_general(at, a1, dn,
                                         preferred_element_type=f32)
                o = jnp.dot(h, w_ref[0], preferred_element_type=f32)
                o = o + jnp.dot(a1, w_ref[1], preferred_element_type=f32)
                o = o + jnp.dot(a2, w_ref[2], preferred_element_type=f32)
                h = _leaky(o + b_ref[...])
                mean = jnp.sum(h, axis=0, keepdims=True) * f32(1.0 / NPG)
                mx = jnp.max(h, axis=0, keepdims=True)
                feats.append(mean)
                feats.append(mx)
            z_ref[g, :, :] = jnp.concatenate(feats, axis=1)
        return carry

    jax.lax.fori_loop(0, G // 2, graph_step, jnp.int32(0))

    inv = one / jnp.sqrt(f32(1.0 + 1e-5))
    z = z_ref[...].reshape(G, NI2) * inv * bng_ref[...] + bnb_ref[...]
    head_cp.wait()
    for i in range(5):
        z = _leaky(jnp.dot(z, lw_vmem[i], preferred_element_type=f32)
                   + lb_ref[i])
    out_ref[...] = jnp.dot(z, ow_ref[...], preferred_element_type=f32) \
        + ob_ref[...]


def kernel(x, batch, params):
    f32 = jnp.float32
    x8 = jnp.pad(x, ((0, 0), (0, 1)))
    pos3 = x[:, :4].reshape(G, NPG, 4)
    post = pos3.transpose(0, 2, 1)
    w1 = jnp.pad(params['conv1_w'], ((0, 0), (0, 1), (0, 0)))

    vmem = lambda: pl.BlockSpec(memory_space=pltpu.VMEM)
    out = pl.pallas_call(
        _body,
        in_specs=[vmem() for _ in range(14)] +
                 [pl.BlockSpec(memory_space=pl.ANY)],
        out_specs=vmem(),
        out_shape=jax.ShapeDtypeStruct((G, NL), f32),
        scratch_shapes=[
            pltpu.VMEM((5, NI2, NI2), f32),
            pltpu.VMEM((N, NPG), f32),
            pltpu.VMEM((G, 1, NI2), f32),
            pltpu.SemaphoreType.DMA,
        ],
    )(
        x8, pos3, post, w1, params['conv1_b'].reshape(1, NI),
        params['conv2_w'], params['conv2_b'].reshape(1, NI),
        params['conv3_w'], params['conv3_b'].reshape(1, NI),
        params['bn_g'].reshape(1, NI2), params['bn_b'].reshape(1, NI2),
        params['lin_b'].reshape(5, 1, NI2), params['out_w'],
        params['out_b'].reshape(1, NL), params['lin_w'],
    )
    return out
